# baseline (speedup 1.0000x reference)
import functools

import jax
import jax.numpy as jnp
from jax import lax
from jax.experimental import pallas as pl
from jax.experimental.pallas import tpu as pltpu
from jax.experimental.pallas import tpu_sc as plsc

QN = 16
DN = 16
KN = 1_000_000
NNBR = 5
NCLS = 32

BLK = 16384
NSPL = 4
QW = BLK // NSPL
GRID = (KN + BLK - 1) // BLK

NEGF = -3.0e38
BIGI = 2**31 - 1


def _topk_extract2(cv, ci, n):
    vs, is_ = [], []
    for _ in range(n):
        m = jnp.max(cv, axis=1, keepdims=True)
        sel = jnp.where(cv == m, ci, BIGI)
        ii = jnp.min(sel, axis=1, keepdims=True)
        vs.append(m)
        is_.append(ii)
        cv = jnp.where(ci == ii, NEGF, cv)
    return jnp.concatenate(vs, axis=1), jnp.concatenate(is_, axis=1)


def _tc_body(q_ref, x_ref, outv_ref, outi_ref, sv_ref, si_ref):
    i = pl.program_id(0)

    @pl.when(i == 0)
    def _init():
        sv_ref[...] = jnp.full((QN, 8), NEGF, jnp.float32)
        si_ref[...] = jnp.full((QN, 8), BIGI, jnp.int32)

    xt = x_ref[...].T
    q2 = q_ref[...] * 2.0
    dots = lax.dot_general(q2.astype(jnp.bfloat16), xt.astype(jnp.bfloat16),
                           (((1,), (0,)), ((), ())),
                           preferred_element_type=jnp.float32)
    xn = jnp.sum(xt * xt, axis=0, keepdims=True)
    neg = dots - xn

    bvs, bis = [sv_ref[:, 0:NNBR]], [si_ref[:, 0:NNBR]]
    for qd in range(NSPL):
        cv = neg[:, qd * QW:(qd + 1) * QW]
        ci = (lax.broadcasted_iota(jnp.int32, (QN, QW), 1)
              + (i * BLK + qd * QW))
        cv = jnp.where(ci < KN, cv, NEGF)
        bv, bi = _topk_extract2(cv, ci, NNBR)
        bvs.append(bv)
        bis.append(bi)
    mv = jnp.concatenate(bvs, axis=1)
    mi = jnp.concatenate(bis, axis=1)
    nv, ni = _topk_extract2(mv, mi, NNBR)
    sv_ref[:, 0:NNBR] = nv
    si_ref[:, 0:NNBR] = ni

    @pl.when(i == GRID - 1)
    def _fin():
        qq = q_ref[...]
        qn = jnp.sum(qq * qq, axis=1, keepdims=True)
        outv_ref[:, 0:NNBR] = sv_ref[:, 0:NNBR] - qn
        outi_ref[:, 0:NNBR] = si_ref[:, 0:NNBR]


def _tc_topk(q, xmat):
    return pl.pallas_call(
        _tc_body,
        grid=(GRID,),
        in_specs=[
            pl.BlockSpec((QN, DN), lambda i: (0, 0)),
            pl.BlockSpec((BLK, DN), lambda i: (i, 0)),
        ],
        out_specs=[
            pl.BlockSpec((QN, 8), lambda i: (0, 0)),
            pl.BlockSpec((QN, 8), lambda i: (0, 0)),
        ],
        out_shape=[
            jax.ShapeDtypeStruct((QN, 8), jnp.float32),
            jax.ShapeDtypeStruct((QN, 8), jnp.int32),
        ],
        scratch_shapes=[
            pltpu.VMEM((QN, 8), jnp.float32),
            pltpu.VMEM((QN, 8), jnp.int32),
        ],
    )(q, xmat)


def _sc_vote_body(labels_hbm, idx_hbm, pred_hbm, idx_v, lab_v, pred_v, sem):
    c = lax.axis_index("c")
    s = lax.axis_index("s")
    wid = s * 2 + c

    @pl.when(wid == 0)
    def _():
        pltpu.sync_copy(idx_hbm, idx_v)
        pltpu.async_copy(labels_hbm.at[idx_v], lab_v, sem).wait()
        labs = [lab_v[pl.ds(j * QN, QN)] for j in range(NNBR)]
        best = jnp.full((QN,), -1, jnp.int32)
        pred = jnp.full((QN,), 0, jnp.int32)
        one = jnp.full((QN,), 1, jnp.int32)
        zero = jnp.full((QN,), 0, jnp.int32)
        for cc in range(NCLS):
            cc_v = jnp.full((QN,), cc, jnp.int32)
            cnt = zero
            for j in range(NNBR):
                cnt = cnt + jnp.where(labs[j] == cc_v, one, zero)
            better = cnt > best
            best = jnp.where(better, cnt, best)
            pred = jnp.where(better, cc_v, pred)
        pred_v[...] = pred
        pltpu.sync_copy(pred_v, pred_hbm)


@functools.cache
def _sc_vote():
    return pl.kernel(
        _sc_vote_body,
        out_type=jax.ShapeDtypeStruct((QN,), jnp.int32),
        mesh=plsc.VectorSubcoreMesh(core_axis_name="c", subcore_axis_name="s"),
        scratch_types=[
            pltpu.VMEM((NNBR * QN,), jnp.int32),
            pltpu.VMEM((NNBR * QN,), jnp.int32),
            pltpu.VMEM((QN,), jnp.int32),
            pltpu.SemaphoreType.DMA,
        ],
    )


def kernel(test_query_embedding, train_x, train_labels):
    q = test_query_embedding
    outv, outi = _tc_topk(q, train_x)
    neg_topk_dist = outv[:, :NNBR]
    idx_flat = outi[:, :NNBR].T.reshape(-1)

    pred = _sc_vote()(train_labels, idx_flat)
    return pred, neg_topk_dist


# speedup vs baseline: 1.0042x; 1.0042x over previous
import functools

import jax
import jax.numpy as jnp
from jax import lax
from jax.experimental import pallas as pl
from jax.experimental.pallas import tpu as pltpu
from jax.experimental.pallas import tpu_sc as plsc

QN = 16
DN = 16
KN = 1_000_000
NNBR = 5
NCLS = 32

BLK = 16384
NSPL = 4
QW = BLK // NSPL
GRID = KN // BLK
TAIL = KN - GRID * BLK
TAILP = 640

NEGF = -3.0e38
BIGI = 2**31 - 1


def _topk_extract2(cv, ci, n):
    vs, is_ = [], []
    for _ in range(n):
        m = jnp.max(cv, axis=1, keepdims=True)
        sel = jnp.where(cv == m, ci, BIGI)
        ii = jnp.min(sel, axis=1, keepdims=True)
        vs.append(m)
        is_.append(ii)
        cv = jnp.where(ci == ii, NEGF, cv)
    return jnp.concatenate(vs, axis=1), jnp.concatenate(is_, axis=1)


def _block_topk(xt, base, limit, sv_ref, si_ref, q2bf, nspl):
    w = xt.shape[1]
    qw = w // nspl
    dots = lax.dot_general(q2bf, xt.astype(jnp.bfloat16),
                           (((1,), (0,)), ((), ())),
                           preferred_element_type=jnp.float32)
    xn = jnp.sum(xt * xt, axis=0, keepdims=True)
    neg = dots - xn

    bvs, bis = [sv_ref[:, 0:NNBR]], [si_ref[:, 0:NNBR]]
    for qd in range(nspl):
        cv = neg[:, qd * qw:(qd + 1) * qw]
        ci = (lax.broadcasted_iota(jnp.int32, (QN, qw), 1) + (base + qd * qw))
        cv = jnp.where(ci < limit, cv, NEGF)
        bv, bi = _topk_extract2(cv, ci, NNBR)
        bvs.append(bv)
        bis.append(bi)
    mv = jnp.concatenate(bvs, axis=1)
    mi = jnp.concatenate(bis, axis=1)
    nv, ni = _topk_extract2(mv, mi, NNBR)
    sv_ref[:, 0:NNBR] = nv
    si_ref[:, 0:NNBR] = ni


def _tc_body(q_ref, x_hbm, outv_ref, outi_ref, sv_ref, si_ref, xtl_ref,
             tsem):
    sv_ref[...] = jnp.full((QN, 8), NEGF, jnp.float32)
    si_ref[...] = jnp.full((QN, 8), BIGI, jnp.int32)
    q2bf = (q_ref[...] * 2.0).astype(jnp.bfloat16)

    tail_cp = pltpu.make_async_copy(
        x_hbm.at[pl.ds(GRID * BLK, TAIL), :], xtl_ref.at[pl.ds(0, TAIL), :],
        tsem)
    tail_cp.start()

    def inner(idxs, x_ref):
        i = idxs[0]
        xt = x_ref[...].T
        _block_topk(xt, i * BLK, KN, sv_ref, si_ref, q2bf, NSPL)

    pltpu.emit_pipeline(
        inner,
        grid=(GRID,),
        in_specs=[pl.BlockSpec((BLK, DN), lambda i: (i, 0))],
        _explicit_indices=True,
    )(x_hbm)

    tail_cp.wait()
    xtt = xtl_ref[...].T
    _block_topk(xtt, GRID * BLK, KN, sv_ref, si_ref, q2bf, 1)

    qq = q_ref[...]
    qn = jnp.sum(qq * qq, axis=1, keepdims=True)
    outv_ref[:, 0:NNBR] = sv_ref[:, 0:NNBR] - qn
    outi_ref[:, 0:NNBR] = si_ref[:, 0:NNBR]


def _tc_topk(q, xmat):
    return pl.pallas_call(
        _tc_body,
        in_specs=[
            pl.BlockSpec(memory_space=pltpu.MemorySpace.VMEM),
            pl.BlockSpec(memory_space=pltpu.MemorySpace.HBM),
        ],
        out_specs=[
            pl.BlockSpec(memory_space=pltpu.MemorySpace.VMEM),
            pl.BlockSpec(memory_space=pltpu.MemorySpace.VMEM),
        ],
        out_shape=[
            jax.ShapeDtypeStruct((QN, 8), jnp.float32),
            jax.ShapeDtypeStruct((QN, 8), jnp.int32),
        ],
        scratch_shapes=[
            pltpu.VMEM((QN, 8), jnp.float32),
            pltpu.VMEM((QN, 8), jnp.int32),
            pltpu.VMEM((TAILP, DN), jnp.float32),
            pltpu.SemaphoreType.DMA,
        ],
    )(q, xmat)


def _sc_vote_body(labels_hbm, idx_hbm, pred_hbm, idx_v, lab_v, pred_v, sem):
    c = lax.axis_index("c")
    s = lax.axis_index("s")
    wid = s * 2 + c

    @pl.when(wid == 0)
    def _():
        pltpu.sync_copy(idx_hbm, idx_v)
        pltpu.async_copy(labels_hbm.at[idx_v], lab_v, sem).wait()
        labs = [lab_v[pl.ds(j * QN, QN)] for j in range(NNBR)]
        best = jnp.full((QN,), -1, jnp.int32)
        pred = jnp.full((QN,), 0, jnp.int32)
        one = jnp.full((QN,), 1, jnp.int32)
        zero = jnp.full((QN,), 0, jnp.int32)
        for cc in range(NCLS):
            cc_v = jnp.full((QN,), cc, jnp.int32)
            cnt = zero
            for j in range(NNBR):
                cnt = cnt + jnp.where(labs[j] == cc_v, one, zero)
            better = cnt > best
            best = jnp.where(better, cnt, best)
            pred = jnp.where(better, cc_v, pred)
        pred_v[...] = pred
        pltpu.sync_copy(pred_v, pred_hbm)


@functools.cache
def _sc_vote():
    return pl.kernel(
        _sc_vote_body,
        out_type=jax.ShapeDtypeStruct((QN,), jnp.int32),
        mesh=plsc.VectorSubcoreMesh(core_axis_name="c", subcore_axis_name="s"),
        scratch_types=[
            pltpu.VMEM((NNBR * QN,), jnp.int32),
            pltpu.VMEM((NNBR * QN,), jnp.int32),
            pltpu.VMEM((QN,), jnp.int32),
            pltpu.SemaphoreType.DMA,
        ],
    )


def kernel(test_query_embedding, train_x, train_labels):
    q = test_query_embedding
    outv, outi = _tc_topk(q, train_x)
    neg_topk_dist = outv[:, :NNBR]
    idx_flat = outi[:, :NNBR].T.reshape(-1)

    pred = _sc_vote()(train_labels, idx_flat)
    return pred, neg_topk_dist
